# baseline (speedup 1.0000x reference)
import jax
import jax.numpy as jnp
from jax import lax
from jax.experimental import pallas as pl
from jax.experimental.pallas import tpu as pltpu
from jax.experimental.pallas import tpu_sc as plsc

STATE = 16384
NCONN = 8192
PDIM = 8192
KBLK = 512

CSPLIT = 8192
SCCOLS = PDIM - CSPLIT
NWORK = 32
NPANEL = 16
NHALF = NWORK // NPANEL
CPW = SCCOLS // NPANEL
RSEG = NCONN // NHALF
ROWB = 256
NRB = RSEG // ROWB
LANES = 16
CV = CPW // LANES

CHUNK = STATE // NWORK


def _mv_body(m_ref, w_ref, o_ref):
    k = pl.program_id(0)

    @pl.when(k == 0)
    def _():
        o_ref[...] = jnp.zeros_like(o_ref)

    m_blk = m_ref[:, pl.ds(k * KBLK, KBLK)]
    o_ref[...] += jax.lax.dot_general(
        m_blk, w_ref[...], (((1,), (0,)), ((), ())),
        preferred_element_type=jnp.float32)


def _matvec_tc(mesa, w):
    out = pl.pallas_call(
        _mv_body,
        grid=(NCONN // KBLK,),
        in_specs=[
            pl.BlockSpec((1, NCONN), lambda k: (0, 0)),
            pl.BlockSpec((KBLK, CSPLIT), lambda k: (k, 0)),
        ],
        out_specs=pl.BlockSpec((1, CSPLIT), lambda k: (0, 0)),
        out_shape=jax.ShapeDtypeStruct((1, CSPLIT), jnp.float32),
    )(mesa.reshape(1, NCONN), w)
    return out.reshape(CSPLIT)


_SPLAT_DNUMS = lax.GatherDimensionNumbers(
    offset_dims=(), collapsed_slice_dims=(0,), start_index_map=(0,))


def _lane_splat(vec, j):
    idx = jnp.full((LANES, 1), j, jnp.int32)
    return lax.gather(vec, idx, _SPLAT_DNUMS, slice_sizes=(1,),
                      mode=lax.GatherScatterMode.PROMISE_IN_BOUNDS)


def _sc_mv_body(mesa_hbm, w_hbm, out_hbm,
                wbuf0, wbuf1, mbuf0, mbuf1, obuf, wsem, msem):
    cid = lax.axis_index("c")
    sid = lax.axis_index("s")
    wid = sid * 2 + cid
    panel = wid // NHALF
    half = wid % NHALF
    c0 = CSPLIT + panel * CPW
    r_base = half * RSEG
    wbufs = (wbuf0, wbuf1)
    mbufs = (mbuf0, mbuf1)

    def start(rb):
        wb, mb = wbufs[rb % 2], mbufs[rb % 2]
        r0 = r_base + rb * ROWB
        cw = pltpu.async_copy(
            w_hbm.at[pl.ds(r0, ROWB), pl.ds(c0, CPW)], wb, wsem)
        cm = pltpu.async_copy(mesa_hbm.at[pl.ds(r0, ROWB)], mb, msem)
        return cw, cm

    def compute(acc, wb, mb):
        def inner(g, a):
            m_vec = mb[pl.ds(g * LANES, LANES)]
            for j in range(LANES):
                mv = _lane_splat(m_vec, j)
                r = g * LANES + j
                a = tuple(
                    a[l] + mv * wb[r, pl.ds(l * LANES, LANES)]
                    for l in range(CV)
                )
            return a

        return lax.fori_loop(0, ROWB // LANES, inner, acc)

    acc = tuple(jnp.zeros((LANES,), jnp.float32) for _ in range(CV))
    pending = start(0)
    for rb in range(NRB):
        nxt = start(rb + 1) if rb + 1 < NRB else None
        pending[0].wait()
        pending[1].wait()
        acc = compute(acc, wbufs[rb % 2], mbufs[rb % 2])
        pending = nxt
    for l in range(CV):
        obuf[pl.ds(l * LANES, LANES)] = acc[l]
    pltpu.sync_copy(obuf, out_hbm.at[pl.ds(half * SCCOLS + panel * CPW, CPW)])


def _matvec_sc(mesa, w):
    run = pl.kernel(
        _sc_mv_body,
        out_type=jax.ShapeDtypeStruct((NHALF * SCCOLS,), jnp.float32),
        mesh=plsc.VectorSubcoreMesh(core_axis_name="c", subcore_axis_name="s"),
        scratch_types=[
            pltpu.VMEM((ROWB, CPW), jnp.float32),
            pltpu.VMEM((ROWB, CPW), jnp.float32),
            pltpu.VMEM((ROWB,), jnp.float32),
            pltpu.VMEM((ROWB,), jnp.float32),
            pltpu.VMEM((CPW,), jnp.float32),
            pltpu.SemaphoreType.DMA,
            pltpu.SemaphoreType.DMA,
        ],
        compiler_params=pltpu.CompilerParams(needs_layout_passes=False),
    )
    return run(mesa, w)


def _sc_body(vtc_hbm, idx_hbm, bias_hbm, out_hbm, idx_v, vals_v,
             buf_v, sem):
    cid = lax.axis_index("c")
    sid = lax.axis_index("s")
    wid = sid * 2 + cid
    base = wid * CHUNK
    c1 = pltpu.async_copy(bias_hbm.at[pl.ds(base, CHUNK)], buf_v, sem)
    c2 = pltpu.async_copy(idx_hbm, idx_v, sem)
    c3 = pltpu.async_copy(vtc_hbm, vals_v.at[pl.ds(0, CSPLIT)], sem)
    c1.wait()
    c2.wait()
    c3.wait()

    UNROLL = 8

    def body(i, carry):
        for u in range(UNROLL):
            off = (i * UNROLL + u) * LANES
            vi = idx_v[pl.ds(off, LANES)]
            vv = vals_v[pl.ds(off, LANES)]
            rel = vi - base
            m = (rel >= 0) & (rel < CHUNK)
            plsc.addupdate_scatter(buf_v, [rel], vv, mask=m)
        return carry

    lax.fori_loop(0, NCONN // (LANES * UNROLL), body, 0)
    pltpu.sync_copy(buf_v, out_hbm.at[pl.ds(base, CHUNK)])


def _sc_scatter(vals_tc, conn_idx, bias):
    run = pl.kernel(
        _sc_body,
        out_type=jax.ShapeDtypeStruct((STATE,), jnp.float32),
        mesh=plsc.VectorSubcoreMesh(core_axis_name="c", subcore_axis_name="s"),
        scratch_types=[
            pltpu.VMEM((NCONN,), jnp.int32),
            pltpu.VMEM((NCONN,), jnp.float32),
            pltpu.VMEM((CHUNK,), jnp.float32),
            pltpu.SemaphoreType.DMA,
        ],
        compiler_params=pltpu.CompilerParams(needs_layout_passes=False),
    )
    return run(vals_tc, conn_idx, bias)


def kernel(mesa_parameter, meta_weight, meta_bias, conn_idx):
    vals_tc = _matvec_tc(mesa_parameter, meta_weight)
    return _sc_scatter(vals_tc, conn_idx, meta_bias)


# speedup vs baseline: 1.0556x; 1.0556x over previous
import jax
import jax.numpy as jnp
from jax import lax
from jax.experimental import pallas as pl
from jax.experimental.pallas import tpu as pltpu
from jax.experimental.pallas import tpu_sc as plsc

STATE = 16384
NCONN = 8192
PDIM = 8192
KBLK = 256

CSPLIT = 8192
SCCOLS = PDIM - CSPLIT
NWORK = 32
NPANEL = 16
NHALF = NWORK // NPANEL
CPW = SCCOLS // NPANEL
RSEG = NCONN // NHALF
ROWB = 256
NRB = RSEG // ROWB
LANES = 16
CV = CPW // LANES

CHUNK = STATE // NWORK


def _mv_body(m_ref, w_ref, o_ref):
    k = pl.program_id(0)

    @pl.when(k == 0)
    def _():
        o_ref[...] = jnp.zeros_like(o_ref)

    m_blk = m_ref[:, pl.ds(k * KBLK, KBLK)]
    o_ref[...] += jax.lax.dot_general(
        m_blk, w_ref[...], (((1,), (0,)), ((), ())),
        preferred_element_type=jnp.float32)


def _matvec_tc(mesa, w):
    out = pl.pallas_call(
        _mv_body,
        grid=(NCONN // KBLK,),
        in_specs=[
            pl.BlockSpec((1, NCONN), lambda k: (0, 0)),
            pl.BlockSpec((KBLK, CSPLIT), lambda k: (k, 0)),
        ],
        out_specs=pl.BlockSpec((1, CSPLIT), lambda k: (0, 0)),
        out_shape=jax.ShapeDtypeStruct((1, CSPLIT), jnp.float32),
    )(mesa.reshape(1, NCONN), w)
    return out.reshape(CSPLIT)


_SPLAT_DNUMS = lax.GatherDimensionNumbers(
    offset_dims=(), collapsed_slice_dims=(0,), start_index_map=(0,))


def _lane_splat(vec, j):
    idx = jnp.full((LANES, 1), j, jnp.int32)
    return lax.gather(vec, idx, _SPLAT_DNUMS, slice_sizes=(1,),
                      mode=lax.GatherScatterMode.PROMISE_IN_BOUNDS)


def _sc_mv_body(mesa_hbm, w_hbm, out_hbm,
                wbuf0, wbuf1, mbuf0, mbuf1, obuf, wsem, msem):
    cid = lax.axis_index("c")
    sid = lax.axis_index("s")
    wid = sid * 2 + cid
    panel = wid // NHALF
    half = wid % NHALF
    c0 = CSPLIT + panel * CPW
    r_base = half * RSEG
    wbufs = (wbuf0, wbuf1)
    mbufs = (mbuf0, mbuf1)

    def start(rb):
        wb, mb = wbufs[rb % 2], mbufs[rb % 2]
        r0 = r_base + rb * ROWB
        cw = pltpu.async_copy(
            w_hbm.at[pl.ds(r0, ROWB), pl.ds(c0, CPW)], wb, wsem)
        cm = pltpu.async_copy(mesa_hbm.at[pl.ds(r0, ROWB)], mb, msem)
        return cw, cm

    def compute(acc, wb, mb):
        def inner(g, a):
            m_vec = mb[pl.ds(g * LANES, LANES)]
            for j in range(LANES):
                mv = _lane_splat(m_vec, j)
                r = g * LANES + j
                a = tuple(
                    a[l] + mv * wb[r, pl.ds(l * LANES, LANES)]
                    for l in range(CV)
                )
            return a

        return lax.fori_loop(0, ROWB // LANES, inner, acc)

    acc = tuple(jnp.zeros((LANES,), jnp.float32) for _ in range(CV))
    pending = start(0)
    for rb in range(NRB):
        nxt = start(rb + 1) if rb + 1 < NRB else None
        pending[0].wait()
        pending[1].wait()
        acc = compute(acc, wbufs[rb % 2], mbufs[rb % 2])
        pending = nxt
    for l in range(CV):
        obuf[pl.ds(l * LANES, LANES)] = acc[l]
    pltpu.sync_copy(obuf, out_hbm.at[pl.ds(half * SCCOLS + panel * CPW, CPW)])


def _matvec_sc(mesa, w):
    run = pl.kernel(
        _sc_mv_body,
        out_type=jax.ShapeDtypeStruct((NHALF * SCCOLS,), jnp.float32),
        mesh=plsc.VectorSubcoreMesh(core_axis_name="c", subcore_axis_name="s"),
        scratch_types=[
            pltpu.VMEM((ROWB, CPW), jnp.float32),
            pltpu.VMEM((ROWB, CPW), jnp.float32),
            pltpu.VMEM((ROWB,), jnp.float32),
            pltpu.VMEM((ROWB,), jnp.float32),
            pltpu.VMEM((CPW,), jnp.float32),
            pltpu.SemaphoreType.DMA,
            pltpu.SemaphoreType.DMA,
        ],
        compiler_params=pltpu.CompilerParams(needs_layout_passes=False),
    )
    return run(mesa, w)


def _sc_body(vtc_hbm, idx_hbm, bias_hbm, out_hbm, idx_v, vals_v,
             buf_v, sem):
    cid = lax.axis_index("c")
    sid = lax.axis_index("s")
    wid = sid * 2 + cid
    base = wid * CHUNK
    c1 = pltpu.async_copy(bias_hbm.at[pl.ds(base, CHUNK)], buf_v, sem)
    c2 = pltpu.async_copy(idx_hbm, idx_v, sem)
    c3 = pltpu.async_copy(vtc_hbm, vals_v.at[pl.ds(0, CSPLIT)], sem)
    c1.wait()
    c2.wait()
    c3.wait()

    ar16 = jnp.arange(LANES, dtype=jnp.int32)
    probe0 = ar16 * (NCONN // LANES)
    v0 = plsc.load_gather(idx_v, [probe0])
    c0 = plsc.all_reduce_population_count(v0 < base)
    b0 = jnp.maximum(c0 - 1, 0) * (NCONN // LANES)
    probe1 = b0 + ar16 * (NCONN // LANES // LANES)
    v1 = plsc.load_gather(idx_v, [probe1])
    c1v = plsc.all_reduce_population_count(v1 < base)
    start_v = b0 + jnp.maximum(c1v - 1, 0) * (NCONN // LANES // LANES)
    NSCAN = CHUNK // LANES + 4
    smax = jnp.max(start_v)
    s0 = smax - lax.rem(smax, LANES)
    s0 = jnp.minimum(s0, NCONN - NSCAN * LANES)

    def body(i, carry):
        for u in range(4):
            off = s0 + (i * 4 + u) * LANES
            vi = idx_v[pl.ds(off, LANES)]
            vv = vals_v[pl.ds(off, LANES)]
            rel = vi - base
            m = (rel >= 0) & (rel < CHUNK)
            plsc.addupdate_scatter(buf_v, [rel], vv, mask=m)
        return carry

    lax.fori_loop(0, NSCAN // 4, body, 0)
    pltpu.sync_copy(buf_v, out_hbm.at[pl.ds(base, CHUNK)])


def _sc_scatter(vals_tc, conn_idx, bias):
    run = pl.kernel(
        _sc_body,
        out_type=jax.ShapeDtypeStruct((STATE,), jnp.float32),
        mesh=plsc.VectorSubcoreMesh(core_axis_name="c", subcore_axis_name="s"),
        scratch_types=[
            pltpu.VMEM((NCONN,), jnp.int32),
            pltpu.VMEM((NCONN,), jnp.float32),
            pltpu.VMEM((CHUNK,), jnp.float32),
            pltpu.SemaphoreType.DMA,
        ],
        compiler_params=pltpu.CompilerParams(needs_layout_passes=False),
    )
    return run(vals_tc, conn_idx, bias)


def kernel(mesa_parameter, meta_weight, meta_bias, conn_idx):
    vals_tc = _matvec_tc(mesa_parameter, meta_weight)
    return _sc_scatter(vals_tc, conn_idx, meta_bias)
